# baseline (speedup 1.0000x reference)
import functools

import jax
import jax.numpy as jnp
from jax import lax
from jax.experimental import pallas as pl
from jax.experimental.pallas import tpu as pltpu
from jax.experimental.pallas import tpu_sc as plsc


def _tt_body(x_ref, out_ref):
  hid, vb = x_ref.shape
  y = jnp.transpose(x_ref[...], (1, 0))
  out_ref[...] = jnp.concatenate([y[:vb // 2, :], y[vb // 2:, :]], axis=1)


def _make_tc_transpose(vocab, hid, vb):
  n_blk = (vocab + vb - 1) // vb
  return pl.pallas_call(
      _tt_body,
      grid=(n_blk,),
      in_specs=[pl.BlockSpec((hid, vb), lambda i: (0, i))],
      out_specs=pl.BlockSpec((vb // 2, 2 * hid), lambda i: (i, 0)),
      out_shape=jax.ShapeDtypeStruct((n_blk * vb // 2, 2 * hid), jnp.float32),
  )


def _make_sc_gather(vocab, hid, n_rows, chunk):
  info = plsc.get_sparse_core_info()
  nc, ns = info.num_cores, info.num_subcores
  nw = nc * ns
  per_w = n_rows // nw
  assert n_rows % nw == 0 and per_w % chunk == 0
  n_chunks = per_w // chunk

  mesh = plsc.VectorSubcoreMesh(core_axis_name="c", subcore_axis_name="s")

  @functools.partial(
      pl.kernel,
      mesh=mesh,
      compiler_params=pltpu.CompilerParams(use_tc_tiling_on_sc=False),
      out_type=jax.ShapeDtypeStruct((n_rows, 2 * hid), jnp.float32),
      scratch_types=[
          pltpu.VMEM((chunk,), jnp.int32),
          pltpu.VMEM((chunk, hid), jnp.float32),
          pltpu.SemaphoreType.DMA,
      ],
  )
  def sc_gather(table_hbm, idx_hbm, out_hbm, idx_v, rows_v, sem):
    wid = lax.axis_index("s") * nc + lax.axis_index("c")
    w_base = wid * per_w

    def body(i, carry):
      base = w_base + i * chunk
      pltpu.sync_copy(idx_hbm.at[pl.ds(base, chunk)], idx_v)
      pltpu.async_copy(table_hbm.at[idx_v], rows_v, sem).wait()
      pltpu.sync_copy(rows_v, out_hbm.at[pl.ds(base, chunk), pl.ds(0, hid)])
      return carry

    lax.fori_loop(0, n_chunks, body, 0)

  return sc_gather


def _conv_body(x_ref, w_ref, b_ref, out_ref):
  bs, s, hid = x_ref.shape
  x = x_ref[...]
  zero = jnp.zeros((bs, 1, hid), jnp.float32)
  x_prev = jnp.concatenate([zero, x[:, :-1, :]], axis=1)
  x_next = jnp.concatenate([x[:, 1:, :], zero], axis=1)
  xcat = jnp.concatenate([x_prev, x, x_next], axis=2)
  y = jnp.dot(
      xcat.reshape(bs * s, 3 * hid), w_ref[...],
      preferred_element_type=jnp.float32)
  y = y.reshape(bs, s, hid)
  m = jnp.max(y, axis=1)
  out_ref[...] = jnp.maximum(m + b_ref[...], 0.0)


def _make_tc_conv(b, s, hid, bs_blk):
  assert b % bs_blk == 0
  grid = (b // bs_blk,)
  return pl.pallas_call(
      _conv_body,
      grid=grid,
      in_specs=[
          pl.BlockSpec((bs_blk, s, hid), lambda i: (i, 0, 0)),
          pl.BlockSpec((3 * hid, hid), lambda i: (0, 0)),
          pl.BlockSpec((1, hid), lambda i: (0, 0)),
      ],
      out_specs=pl.BlockSpec((bs_blk, hid), lambda i: (i, 0)),
      out_shape=jax.ShapeDtypeStruct((b, hid), jnp.float32),
  )


def kernel(input, table, conv_w, conv_b):
  b, s = input.shape
  vocab, hid = table.shape
  k = conv_w.shape[2]
  n_rows = b * s

  idx = input.reshape(n_rows)

  vb = 32768
  tc_t = _make_tc_transpose(vocab, hid, vb=vb)
  packed = tc_t(jnp.transpose(table, (1, 0)))
  table_rows = packed.reshape(packed.shape[0] * 2, hid)

  j = idx % vb
  idx2 = (idx // vb) * vb + (j % (vb // 2)) * 2 + j // (vb // 2)

  sc_gather = _make_sc_gather(vocab, hid, n_rows, chunk=1600)
  out_wide = sc_gather(table_rows, idx2)
  embed = out_wide.reshape(b, s, 2 * hid)[:, :, :hid]

  w_full = jnp.transpose(conv_w, (2, 1, 0)).reshape(k * hid, hid)
  tc_conv = _make_tc_conv(b, s, hid, bs_blk=128)
  hidden = tc_conv(embed, w_full, conv_b.reshape(1, hid))

  return (embed, hidden)
